# baseline (speedup 1.0000x reference)
import functools

import jax
import jax.numpy as jnp
from jax import lax
from jax.experimental import pallas as pl
from jax.experimental.pallas import tpu as pltpu
from jax.experimental.pallas import tpu_sc as plsc

_N = 16384
_C = 1000
_K = 4096
_INT_MIN = -2147483648

_NW = 32
_ROWS_W = _N // _NW
_CH = 16
_NCH = _ROWS_W // _CH
_CHW = _CH * _C
_FULL = _C // 16
_TAIL = _C - _FULL * 16


def _row_sums(buf, tgtb, sumsb, xtb, chunk):
    lanes = lax.iota(jnp.int32, 16)
    tailmask = lanes < _TAIL

    def row_body(rr, res):
        base = rr * _C
        accs = [jnp.exp(buf[pl.ds(base + c * 16, 16)]) for c in range(8)]
        for c in range(8, _FULL):
            accs[c % 8] = accs[c % 8] + jnp.exp(buf[pl.ds(base + c * 16, 16)])
        tail = buf[pl.ds(base + _FULL * 16, 16)]
        accs[_FULL % 8] = accs[_FULL % 8] + jnp.where(
            tailmask, jnp.exp(tail), jnp.float32(0.0))
        acc = ((accs[0] + accs[1]) + (accs[2] + accs[3])) + (
            (accs[4] + accs[5]) + (accs[6] + accs[7]))
        s = jnp.sum(acc)
        return jnp.where(lanes == rr, s, res)

    res = lax.fori_loop(0, _CH, row_body, jnp.zeros((16,), jnp.float32))
    sumsb[pl.ds(chunk * _CH, 16)] = res
    t_vec = tgtb[pl.ds(chunk * _CH, 16)]
    addr = lanes * _C + t_vec
    xtb[pl.ds(chunk * _CH, 16)] = plsc.load_gather(buf, [addr])


def _sc_ce(x_hbm, tgt_hbm, sums_out, xt_out,
           buf0, buf1, sumsb, xtb, tgtb, sem0, sem1):
    wid = lax.axis_index("s") * 2 + lax.axis_index("c")
    row0 = wid * _ROWS_W
    base = row0 * _C

    pltpu.sync_copy(tgt_hbm.at[pl.ds(row0, _ROWS_W)], tgtb)

    def chunk_src(g):
        gg = jnp.minimum(g, _NCH - 1)
        return x_hbm.at[pl.ds(base + gg * _CHW, _CHW)]

    pltpu.async_copy(chunk_src(jnp.int32(0)), buf0.at[pl.ds(0, _CHW)], sem0)
    pltpu.async_copy(chunk_src(jnp.int32(1)), buf1.at[pl.ds(0, _CHW)], sem1)

    def outer(g, carry):
        g0 = g * 2
        pltpu.make_async_copy(chunk_src(g0), buf0.at[pl.ds(0, _CHW)], sem0).wait()
        _row_sums(buf0, tgtb, sumsb, xtb, g0)
        pltpu.async_copy(chunk_src(g0 + 2), buf0.at[pl.ds(0, _CHW)], sem0)

        g1 = g0 + 1
        pltpu.make_async_copy(chunk_src(g1), buf1.at[pl.ds(0, _CHW)], sem1).wait()
        _row_sums(buf1, tgtb, sumsb, xtb, g1)
        pltpu.async_copy(chunk_src(g1 + 2), buf1.at[pl.ds(0, _CHW)], sem1)
        return carry

    lax.fori_loop(0, _NCH // 2, outer, jnp.int32(0))

    pltpu.make_async_copy(chunk_src(jnp.int32(_NCH - 1)),
                          buf0.at[pl.ds(0, _CHW)], sem0).wait()
    pltpu.make_async_copy(chunk_src(jnp.int32(_NCH - 1)),
                          buf1.at[pl.ds(0, _CHW)], sem1).wait()

    pltpu.sync_copy(sumsb, sums_out.at[pl.ds(row0, _ROWS_W)])
    pltpu.sync_copy(xtb, xt_out.at[pl.ds(row0, _ROWS_W)])


def _select_body(s_ref, xt_ref, out_ref):
    L = jnp.log(s_ref[...]) - xt_ref[...]
    b = lax.bitcast_convert_type(L, jnp.int32)
    keys = jnp.where(b < 0, jnp.bitwise_not(b) ^ jnp.int32(_INT_MIN), b)

    def step(i, tu):
        bit = jnp.int32(31) - i
        cand = tu | (jnp.int32(1) << bit)
        cand_s = cand ^ jnp.int32(_INT_MIN)
        cnt = jnp.sum((keys >= cand_s).astype(jnp.int32))
        return jnp.where(cnt >= _K, cand, tu)

    tu = lax.fori_loop(0, 32, step, jnp.int32(0))
    tu_s = tu ^ jnp.int32(_INT_MIN)
    tb = jnp.where(tu < 0, tu ^ jnp.int32(_INT_MIN), jnp.bitwise_not(tu))
    tval = lax.bitcast_convert_type(tb, jnp.float32)

    gt = keys > tu_s
    cnt_gt = jnp.sum(gt.astype(jnp.float32))
    sum_gt = jnp.sum(jnp.where(gt, L, 0.0))
    res = (sum_gt + (jnp.float32(_K) - cnt_gt) * tval) / _K
    out_ref[...] = res.reshape(1, 1)


@functools.partial(jax.jit)
def kernel(inputs, targets):
    mesh = plsc.VectorSubcoreMesh(core_axis_name="c", subcore_axis_name="s")
    sc_ce = functools.partial(
        pl.kernel,
        mesh=mesh,
        compiler_params=pltpu.CompilerParams(needs_layout_passes=False),
        out_type=[
            jax.ShapeDtypeStruct((_N,), jnp.float32),
            jax.ShapeDtypeStruct((_N,), jnp.float32),
        ],
        scratch_types=[
            pltpu.VMEM((_CHW + 16,), jnp.float32),
            pltpu.VMEM((_CHW + 16,), jnp.float32),
            pltpu.VMEM((_ROWS_W,), jnp.float32),
            pltpu.VMEM((_ROWS_W,), jnp.float32),
            pltpu.VMEM((_ROWS_W,), jnp.int32),
            pltpu.SemaphoreType.DMA,
            pltpu.SemaphoreType.DMA,
        ],
    )(_sc_ce)
    sums, xt = sc_ce(inputs.reshape(_N * _C), targets)

    out = pl.pallas_call(
        _select_body,
        in_specs=[
            pl.BlockSpec((128, 128), lambda: (0, 0)),
            pl.BlockSpec((128, 128), lambda: (0, 0)),
        ],
        out_specs=pl.BlockSpec((1, 1), lambda: (0, 0)),
        out_shape=jax.ShapeDtypeStruct((1, 1), jnp.float32),
    )(sums.reshape(128, 128), xt.reshape(128, 128))
    return out[0, 0]


# speedup vs baseline: 1.5975x; 1.5975x over previous
import functools

import jax
import jax.numpy as jnp
from jax import lax
from jax.experimental import pallas as pl
from jax.experimental.pallas import tpu as pltpu
from jax.experimental.pallas import tpu_sc as plsc

_N = 16384
_C = 1000
_K = 4096
_INT_MIN = -2147483648

_NW = 32
_ROWS_W = _N // _NW
_CH = 16
_NCH = _ROWS_W // _CH
_CHW = _CH * _C
_FULL = _C // 16
_TAIL = _C - _FULL * 16


def _row_sums(buf, tgtb, sumsb, xtb, chunk):
    lanes = lax.iota(jnp.int32, 16)
    tailmask = lanes >= (16 - _TAIL)

    def row_body(rr, res):
        accs = [jnp.exp(buf[rr, pl.ds(c * 16, 16)]) for c in range(8)]
        for c in range(8, _FULL):
            accs[c % 8] = accs[c % 8] + jnp.exp(buf[rr, pl.ds(c * 16, 16)])
        tail = buf[rr, pl.ds(_C - 16, 16)]
        accs[_FULL % 8] = accs[_FULL % 8] + jnp.where(
            tailmask, jnp.exp(tail), jnp.float32(0.0))
        acc = ((accs[0] + accs[1]) + (accs[2] + accs[3])) + (
            (accs[4] + accs[5]) + (accs[6] + accs[7]))
        s = jnp.sum(acc)
        return jnp.where(lanes == rr, s, res)

    res = lax.fori_loop(0, _CH, row_body, jnp.zeros((16,), jnp.float32))
    sumsb[pl.ds(chunk * _CH, 16)] = res
    t_vec = tgtb[pl.ds(chunk * _CH, 16)]
    xtb[pl.ds(chunk * _CH, 16)] = plsc.load_gather(buf, [lanes, t_vec])


def _sc_ce(x_hbm, tgt_hbm, sums_out, xt_out,
           buf0, buf1, sumsb, xtb, tgtb, sem0, sem1):
    wid = lax.axis_index("s") * 2 + lax.axis_index("c")
    row0 = wid * _ROWS_W

    pltpu.sync_copy(tgt_hbm.at[pl.ds(row0, _ROWS_W)], tgtb)

    def chunk_src(g):
        gg = jnp.minimum(g, _NCH - 1)
        return x_hbm.at[pl.ds(row0 + gg * _CH, _CH), :]

    pltpu.async_copy(chunk_src(jnp.int32(0)), buf0, sem0)
    pltpu.async_copy(chunk_src(jnp.int32(1)), buf1, sem1)

    def outer(g, carry):
        g0 = g * 2
        pltpu.make_async_copy(chunk_src(g0), buf0, sem0).wait()
        _row_sums(buf0, tgtb, sumsb, xtb, g0)
        pltpu.async_copy(chunk_src(g0 + 2), buf0, sem0)

        g1 = g0 + 1
        pltpu.make_async_copy(chunk_src(g1), buf1, sem1).wait()
        _row_sums(buf1, tgtb, sumsb, xtb, g1)
        pltpu.async_copy(chunk_src(g1 + 2), buf1, sem1)
        return carry

    lax.fori_loop(0, _NCH // 2, outer, jnp.int32(0))

    pltpu.make_async_copy(chunk_src(jnp.int32(_NCH - 1)), buf0, sem0).wait()
    pltpu.make_async_copy(chunk_src(jnp.int32(_NCH - 1)), buf1, sem1).wait()

    pltpu.sync_copy(sumsb, sums_out.at[pl.ds(row0, _ROWS_W)])
    pltpu.sync_copy(xtb, xt_out.at[pl.ds(row0, _ROWS_W)])


def _select_body(s_ref, xt_ref, out_ref):
    L = jnp.log(s_ref[...]) - xt_ref[...]
    b = lax.bitcast_convert_type(L, jnp.int32)
    keys = jnp.where(b < 0, jnp.bitwise_not(b) ^ jnp.int32(_INT_MIN), b)

    def step(i, tu):
        bit = jnp.int32(31) - i
        cand = tu | (jnp.int32(1) << bit)
        cand_s = cand ^ jnp.int32(_INT_MIN)
        cnt = jnp.sum((keys >= cand_s).astype(jnp.int32))
        return jnp.where(cnt >= _K, cand, tu)

    tu = lax.fori_loop(0, 32, step, jnp.int32(0))
    tu_s = tu ^ jnp.int32(_INT_MIN)
    tb = jnp.where(tu < 0, tu ^ jnp.int32(_INT_MIN), jnp.bitwise_not(tu))
    tval = lax.bitcast_convert_type(tb, jnp.float32)

    gt = keys > tu_s
    cnt_gt = jnp.sum(gt.astype(jnp.float32))
    sum_gt = jnp.sum(jnp.where(gt, L, 0.0))
    res = (sum_gt + (jnp.float32(_K) - cnt_gt) * tval) / _K
    out_ref[...] = res.reshape(1, 1)


@functools.partial(jax.jit)
def kernel(inputs, targets):
    mesh = plsc.VectorSubcoreMesh(core_axis_name="c", subcore_axis_name="s")
    sc_ce = functools.partial(
        pl.kernel,
        mesh=mesh,
        compiler_params=pltpu.CompilerParams(needs_layout_passes=False),
        out_type=[
            jax.ShapeDtypeStruct((_N,), jnp.float32),
            jax.ShapeDtypeStruct((_N,), jnp.float32),
        ],
        scratch_types=[
            pltpu.VMEM((_CH, _C), jnp.float32),
            pltpu.VMEM((_CH, _C), jnp.float32),
            pltpu.VMEM((_ROWS_W,), jnp.float32),
            pltpu.VMEM((_ROWS_W,), jnp.float32),
            pltpu.VMEM((_ROWS_W,), jnp.int32),
            pltpu.SemaphoreType.DMA,
            pltpu.SemaphoreType.DMA,
        ],
    )(_sc_ce)
    sums, xt = sc_ce(inputs, targets)

    out = pl.pallas_call(
        _select_body,
        in_specs=[
            pl.BlockSpec((128, 128), lambda: (0, 0)),
            pl.BlockSpec((128, 128), lambda: (0, 0)),
        ],
        out_specs=pl.BlockSpec((1, 1), lambda: (0, 0)),
        out_shape=jax.ShapeDtypeStruct((1, 1), jnp.float32),
    )(sums.reshape(128, 128), xt.reshape(128, 128))
    return out[0, 0]


# speedup vs baseline: 1.7890x; 1.1199x over previous
import functools

import jax
import jax.numpy as jnp
from jax import lax
from jax.experimental import pallas as pl
from jax.experimental.pallas import tpu as pltpu
from jax.experimental.pallas import tpu_sc as plsc

_N = 16384
_C = 1000
_K = 4096
_INT_MIN = -2147483648

_N_SC = 8192
_N_TC = _N - _N_SC
_BLK = 1024
_TC_GRID = _N_TC // _BLK

_NW = 32
_ROWS_W = _N_SC // _NW
_CH = 16
_NCH = _ROWS_W // _CH
_FULL = _C // 16
_TAIL = _C - _FULL * 16


def _row_sums(buf, tgtb, sumsb, xtb, chunk):
    lanes = lax.iota(jnp.int32, 16)
    tailmask = lanes >= (16 - _TAIL)

    def row_body(rr, res):
        accs = [jnp.exp(buf[rr, pl.ds(c * 16, 16)]) for c in range(8)]
        for c in range(8, _FULL):
            accs[c % 8] = accs[c % 8] + jnp.exp(buf[rr, pl.ds(c * 16, 16)])
        tail = buf[rr, pl.ds(_C - 16, 16)]
        accs[_FULL % 8] = accs[_FULL % 8] + jnp.where(
            tailmask, jnp.exp(tail), jnp.float32(0.0))
        acc = ((accs[0] + accs[1]) + (accs[2] + accs[3])) + (
            (accs[4] + accs[5]) + (accs[6] + accs[7]))
        s = jnp.sum(acc)
        return jnp.where(lanes == rr, s, res)

    res = lax.fori_loop(0, _CH, row_body, jnp.zeros((16,), jnp.float32))
    sumsb[pl.ds(chunk * _CH, 16)] = res
    t_vec = tgtb[pl.ds(chunk * _CH, 16)]
    xtb[pl.ds(chunk * _CH, 16)] = plsc.load_gather(buf, [lanes, t_vec])


def _sc_ce(x_hbm, tgt_hbm, sums_out, xt_out,
           buf0, buf1, sumsb, xtb, tgtb, sem0, sem1):
    wid = lax.axis_index("s") * 2 + lax.axis_index("c")
    row0 = wid * _ROWS_W

    pltpu.sync_copy(tgt_hbm.at[pl.ds(row0, _ROWS_W)], tgtb)

    def chunk_src(g):
        gg = jnp.minimum(g, _NCH - 1)
        return x_hbm.at[pl.ds(row0 + gg * _CH, _CH), :]

    pltpu.async_copy(chunk_src(jnp.int32(0)), buf0, sem0)
    pltpu.async_copy(chunk_src(jnp.int32(1)), buf1, sem1)

    def outer(g, carry):
        g0 = g * 2
        pltpu.make_async_copy(chunk_src(g0), buf0, sem0).wait()
        _row_sums(buf0, tgtb, sumsb, xtb, g0)
        pltpu.async_copy(chunk_src(g0 + 2), buf0, sem0)

        g1 = g0 + 1
        pltpu.make_async_copy(chunk_src(g1), buf1, sem1).wait()
        _row_sums(buf1, tgtb, sumsb, xtb, g1)
        pltpu.async_copy(chunk_src(g1 + 2), buf1, sem1)
        return carry

    lax.fori_loop(0, _NCH // 2, outer, jnp.int32(0))

    pltpu.make_async_copy(chunk_src(jnp.int32(_NCH - 1)), buf0, sem0).wait()
    pltpu.make_async_copy(chunk_src(jnp.int32(_NCH - 1)), buf1, sem1).wait()

    pltpu.sync_copy(sumsb, sums_out.at[pl.ds(row0, _ROWS_W)])
    pltpu.sync_copy(xtb, xt_out.at[pl.ds(row0, _ROWS_W)])


def _tc_ce_body(x_ref, t_ref, loss_ref):
    x = x_ref[...]
    t = t_ref[0, 0, :]
    m = jnp.max(x, axis=1, keepdims=True)
    e = jnp.exp(x - m)
    logz = m[:, 0] + jnp.log(jnp.sum(e, axis=1))
    cols = lax.broadcasted_iota(jnp.int32, (_BLK, _C), 1)
    tgt = jnp.sum(jnp.where(cols == t[:, None], x, 0.0), axis=1)
    loss = logz - tgt
    loss_ref[...] = loss.reshape(_BLK // 128, 128)


def _select_body(tl_ref, s_ref, xt_ref, out_ref):
    L1 = tl_ref[...]
    L2 = jnp.log(s_ref[...]) - xt_ref[...]

    def keys_of(L):
        b = lax.bitcast_convert_type(L, jnp.int32)
        return jnp.where(b < 0, jnp.bitwise_not(b) ^ jnp.int32(_INT_MIN), b)

    k1, k2 = keys_of(L1), keys_of(L2)

    def step(i, tu):
        bit = jnp.int32(31) - i
        cand = tu | (jnp.int32(1) << bit)
        cand_s = cand ^ jnp.int32(_INT_MIN)
        cnt = (jnp.sum((k1 >= cand_s).astype(jnp.int32))
               + jnp.sum((k2 >= cand_s).astype(jnp.int32)))
        return jnp.where(cnt >= _K, cand, tu)

    tu = lax.fori_loop(0, 32, step, jnp.int32(0))
    tu_s = tu ^ jnp.int32(_INT_MIN)
    tb = jnp.where(tu < 0, tu ^ jnp.int32(_INT_MIN), jnp.bitwise_not(tu))
    tval = lax.bitcast_convert_type(tb, jnp.float32)

    g1, g2 = k1 > tu_s, k2 > tu_s
    cnt_gt = (jnp.sum(g1.astype(jnp.float32))
              + jnp.sum(g2.astype(jnp.float32)))
    sum_gt = (jnp.sum(jnp.where(g1, L1, 0.0))
              + jnp.sum(jnp.where(g2, L2, 0.0)))
    res = (sum_gt + (jnp.float32(_K) - cnt_gt) * tval) / _K
    out_ref[...] = res.reshape(1, 1)


@functools.partial(jax.jit)
def kernel(inputs, targets):
    mesh = plsc.VectorSubcoreMesh(core_axis_name="c", subcore_axis_name="s")
    sc_ce = functools.partial(
        pl.kernel,
        mesh=mesh,
        compiler_params=pltpu.CompilerParams(needs_layout_passes=False),
        out_type=[
            jax.ShapeDtypeStruct((_N_SC,), jnp.float32),
            jax.ShapeDtypeStruct((_N_SC,), jnp.float32),
        ],
        scratch_types=[
            pltpu.VMEM((_CH, _C), jnp.float32),
            pltpu.VMEM((_CH, _C), jnp.float32),
            pltpu.VMEM((_ROWS_W,), jnp.float32),
            pltpu.VMEM((_ROWS_W,), jnp.float32),
            pltpu.VMEM((_ROWS_W,), jnp.int32),
            pltpu.SemaphoreType.DMA,
            pltpu.SemaphoreType.DMA,
        ],
    )(_sc_ce)
    sums, xt = sc_ce(inputs, targets)

    t3 = targets.reshape(_N // _BLK, 1, _BLK)
    tc_loss = pl.pallas_call(
        _tc_ce_body,
        grid=(_TC_GRID,),
        in_specs=[
            pl.BlockSpec((_BLK, _C), lambda j: (j + _N_SC // _BLK, 0)),
            pl.BlockSpec((1, 1, _BLK), lambda j: (j + _N_SC // _BLK, 0, 0)),
        ],
        out_specs=pl.BlockSpec((_BLK // 128, 128), lambda j: (j, 0)),
        out_shape=jax.ShapeDtypeStruct((_N_TC // 128, 128), jnp.float32),
    )(inputs, t3)

    out = pl.pallas_call(
        _select_body,
        in_specs=[
            pl.BlockSpec((_N_TC // 128, 128), lambda: (0, 0)),
            pl.BlockSpec((_N_SC // 128, 128), lambda: (0, 0)),
            pl.BlockSpec((_N_SC // 128, 128), lambda: (0, 0)),
        ],
        out_specs=pl.BlockSpec((1, 1), lambda: (0, 0)),
        out_shape=jax.ShapeDtypeStruct((1, 1), jnp.float32),
    )(tc_loss, sums.reshape(_N_SC // 128, 128), xt.reshape(_N_SC // 128, 128))
    return out[0, 0]


# speedup vs baseline: 2.1387x; 1.1955x over previous
import functools

import jax
import jax.numpy as jnp
from jax.experimental import pallas as pl
from jax.experimental.pallas import tpu as pltpu

_N = 16384
_C = 1000
_BLK = 2048
_GRID = _N // _BLK
_K = 4096
_INT_MIN = -2147483648


def _ohem_body(x_ref, t_ref, out_ref, loss_ref):
    j = pl.program_id(0)

    x = x_ref[...]
    t = t_ref[0, 0, :]

    m = jnp.max(x, axis=1, keepdims=True)
    e = jnp.exp(x - m)
    logz = m[:, 0] + jnp.log(jnp.sum(e, axis=1))
    cols = jax.lax.broadcasted_iota(jnp.int32, (_BLK, _C), 1)
    tgt = jnp.sum(jnp.where(cols == t[:, None], x, 0.0), axis=1)
    loss = logz - tgt

    rows = _BLK // 128
    loss_ref[pl.ds(j * rows, rows), :] = loss.reshape(rows, 128)

    @pl.when(j == _GRID - 1)
    def _select():
        L = loss_ref[...]
        b = jax.lax.bitcast_convert_type(L, jnp.int32)
        keys = jnp.where(b < 0, jnp.bitwise_not(b) ^ jnp.int32(_INT_MIN), b)

        def step(i, tu):
            p = jnp.int32(30) - 2 * i
            c1 = tu | (jnp.int32(1) << p)
            c2 = tu | (jnp.int32(2) << p)
            c3 = tu | (jnp.int32(3) << p)
            n1 = jnp.sum((keys >= (c1 ^ jnp.int32(_INT_MIN))).astype(jnp.int32))
            n2 = jnp.sum((keys >= (c2 ^ jnp.int32(_INT_MIN))).astype(jnp.int32))
            n3 = jnp.sum((keys >= (c3 ^ jnp.int32(_INT_MIN))).astype(jnp.int32))
            add = jnp.where(n3 >= _K, jnp.int32(3),
                            jnp.where(n2 >= _K, jnp.int32(2),
                                      jnp.where(n1 >= _K, jnp.int32(1),
                                                jnp.int32(0))))
            return tu | (add << p)

        tu = jax.lax.fori_loop(0, 16, step, jnp.int32(0))
        tu_s = tu ^ jnp.int32(_INT_MIN)
        tb = jnp.where(tu < 0, tu ^ jnp.int32(_INT_MIN), jnp.bitwise_not(tu))
        tval = jax.lax.bitcast_convert_type(tb, jnp.float32)

        gt = keys > tu_s
        cnt_gt = jnp.sum(gt.astype(jnp.float32))
        sum_gt = jnp.sum(jnp.where(gt, L, 0.0))
        res = (sum_gt + (jnp.float32(_K) - cnt_gt) * tval) / _K
        out_ref[...] = res.reshape(1, 1)


@functools.partial(jax.jit)
def kernel(inputs, targets):
    t3 = targets.reshape(_GRID, 1, _BLK)
    out = pl.pallas_call(
        _ohem_body,
        grid=(_GRID,),
        in_specs=[
            pl.BlockSpec((_BLK, _C), lambda j: (j, 0)),
            pl.BlockSpec((1, 1, _BLK), lambda j: (j, 0, 0)),
        ],
        out_specs=pl.BlockSpec((1, 1), lambda j: (0, 0)),
        out_shape=jax.ShapeDtypeStruct((1, 1), jnp.float32),
        scratch_shapes=[pltpu.VMEM((128, 128), jnp.float32)],
    )(inputs, t3)
    return out[0, 0]
